# baseline (speedup 1.0000x reference)
import functools

import jax
import jax.numpy as jnp
from jax import lax
from jax.experimental import pallas as pl
from jax.experimental.pallas import tpu as pltpu
from jax.experimental.pallas import tpu_sc as plsc

N = 10000
T = 3
E = 320000
XD = 128
HD = 256
ZD = 128
NPAD = 10240
DEGR = T * NPAD
BN = 400
NB = N // BN

_mesh = plsc.VectorSubcoreMesh(core_axis_name="c", subcore_axis_name="s")


@functools.partial(
    pl.kernel, mesh=_mesh,
    out_type=jax.ShapeDtypeStruct((2 * NPAD, 128), jnp.float32),
    scratch_types=[
        pltpu.VMEM((80,), jnp.int32),
        pltpu.VMEM((80, 128), jnp.float32),
        pltpu.VMEM((128, 128), jnp.float32),
        pltpu.VMEM_SHARED((NPAD, 128), jnp.float32),
    ],
)
def _deg_kernel(dst_hbm, out_hbm, didx, ones_rows, zbuf, shared):
    c = lax.axis_index("c")
    s = lax.axis_index("s")
    w = s * 2 + c

    def zrow(r, carry):
        for j in range(8):
            zbuf[r, pl.ds(16 * j, 16)] = jnp.zeros((16,), jnp.float32)
        return carry
    lax.fori_loop(0, 128, zrow, 0)

    def orow(r, carry):
        for j in range(8):
            ones_rows[r, pl.ds(16 * j, 16)] = jnp.ones((16,), jnp.float32)
        return carry
    lax.fori_loop(0, 80, orow, 0)

    def zshared(i, carry):
        k = s + i * 16
        pltpu.sync_copy(zbuf, shared.at[pl.ds(k * 128, 128)])
        return carry
    lax.fori_loop(0, NPAD // 128 // 16, zshared, 0)
    plsc.subcore_barrier()

    def chunk(k, carry):
        base = w * (E // 32) + k * 80
        pltpu.sync_copy(dst_hbm.at[pl.ds(base, 80)], didx)
        pltpu.sync_copy(ones_rows, shared.at[didx], add=True)
        return carry
    lax.fori_loop(0, (E // 32) // 80, chunk, 0)
    plsc.subcore_barrier()

    def cpout(i, carry):
        k = s + i * 16
        pltpu.sync_copy(shared.at[pl.ds(k * 128, 128)], zbuf)
        pltpu.sync_copy(zbuf, out_hbm.at[pl.ds(c * NPAD + k * 128, 128)])
        return carry
    lax.fori_loop(0, NPAD // 128 // 16, cpout, 0)


def _make_agg(split_features):
    nch = 160 if split_features else 80
    SEG = 16 if split_features else 8
    NSEG = nch // SEG
    CPR = 80
    NCH = N // CPR

    @functools.partial(
        pl.kernel, mesh=_mesh,
        out_type=jax.ShapeDtypeStruct((2 * N, 128), jnp.float32),
        scratch_types=[
            pltpu.VMEM((SEG, 128), jnp.int32),
            pltpu.VMEM((SEG, 128), jnp.int32),
            pltpu.VMEM((128, 128), jnp.float32),
            pltpu.VMEM_SHARED((N + 16, 128), jnp.float32),
            pltpu.SemaphoreType.DMA,
        ],
    )
    def _agg(y_hbm, src_hbm, dst_hbm, out_hbm, sidx, didx,
             rows_a, shared, sem_a):
        c = lax.axis_index("c")
        s = lax.axis_index("s")
        if split_features:
            rowbase = c * (16 * nch) + s * nch
        else:
            rowbase = (s * 2 + c) * nch

        def zrow(r, carry):
            for j in range(8):
                rows_a[r, pl.ds(16 * j, 16)] = jnp.zeros((16,), jnp.float32)
            return carry
        lax.fori_loop(0, 128, zrow, 0)

        nmine = jnp.where(s < (NCH % 16), (NCH // 16) + 1, NCH // 16)

        def zshared(i, carry):
            k = s + i * 16
            pltpu.sync_copy(rows_a.at[pl.ds(0, CPR)], shared.at[pl.ds(k * CPR, CPR)])
            return carry
        lax.fori_loop(0, nmine, zshared, 0)

        plsc.subcore_barrier()

        def seg(j, carry):
            row0 = rowbase + j * SEG
            pltpu.sync_copy(src_hbm.at[pl.ds(row0, SEG)], sidx)
            pltpu.sync_copy(dst_hbm.at[pl.ds(row0, SEG)], didx)
            for k in range(SEG):
                pltpu.async_copy(y_hbm.at[sidx.at[k]], rows_a, sem_a).wait()
                pltpu.sync_copy(rows_a, shared.at[didx.at[k]], add=True)
            return carry
        lax.fori_loop(0, NSEG, seg, 0)
        plsc.subcore_barrier()

        def cpout(i, carry):
            k = s + i * 16
            pltpu.sync_copy(shared.at[pl.ds(k * CPR, CPR)], rows_a.at[pl.ds(0, CPR)])
            pltpu.sync_copy(rows_a.at[pl.ds(0, CPR)],
                            out_hbm.at[pl.ds(c * N + k * CPR, CPR)])
            return carry
        lax.fori_loop(0, nmine, cpout, 0)

    return _agg


_agg1 = _make_agg(True)
_agg2 = _make_agg(False)


def _prep1(src, dst):
    s16 = src.reshape(16, E // 16)
    padi = jnp.zeros((16, 480), jnp.int32)
    padd = N + jnp.broadcast_to(jnp.arange(16, dtype=jnp.int32)[:, None], (16, 480))
    s0 = jnp.concatenate([s16, padi], 1)
    s1 = jnp.concatenate([s16 + N, padi], 1)
    src2d = jnp.concatenate([s0, s1], 0).reshape(2 * 16 * 160, 128)
    d16 = jnp.concatenate([dst.reshape(16, E // 16), padd], 1)
    dst2d = jnp.concatenate([d16, d16], 0).reshape(2 * 16 * 160, 128)
    return src2d, dst2d


def _prep2(src, dst):
    padi = jnp.zeros((32, 240), jnp.int32)
    padd = N + jnp.broadcast_to(
        (jnp.arange(32, dtype=jnp.int32) // 2)[:, None], (32, 240))
    src2d = jnp.concatenate([src.reshape(32, E // 32), padi], 1).reshape(32 * 80, 128)
    dst2d = jnp.concatenate([dst.reshape(32, E // 32), padd], 1).reshape(32 * 80, 128)
    return src2d, dst2d


def _x_body(xs_ref, w_ref, b_ref, o_ref):
    o_ref[...] = jnp.maximum(
        jnp.dot(xs_ref[...], w_ref[...], preferred_element_type=jnp.float32)
        + b_ref[...], 0.0)


def _compute_x(xs, W, b):
    return pl.pallas_call(
        _x_body,
        grid=(NB,),
        in_specs=[pl.BlockSpec((BN, XD), lambda i: (i, 0)),
                  pl.BlockSpec((XD, HD), lambda i: (0, 0)),
                  pl.BlockSpec((1, HD), lambda i: (0, 0))],
        out_specs=pl.BlockSpec((BN, HD), lambda i: (i, 0)),
        out_shape=jax.ShapeDtypeStruct((N, HD), jnp.float32),
    )(xs, W, b.reshape(1, HD))


_RB = 1280


def _dinv_body(p_ref, o_ref):
    d = p_ref[0] + p_ref[1]
    deg = 1.0 + jnp.sum(d, axis=1) * (1.0 / 128.0)
    o_ref[...] = lax.rsqrt(deg)[:, None]


def _compute_dinv(parts):
    return pl.pallas_call(
        _dinv_body,
        grid=(NPAD // _RB,),
        in_specs=[pl.BlockSpec((2, _RB, 128), lambda i: (0, i, 0))],
        out_specs=pl.BlockSpec((_RB, 1), lambda i: (i, 0)),
        out_shape=jax.ShapeDtypeStruct((NPAD, 1), jnp.float32),
    )(parts)


def _y1_body(x_ref, h_ref, w_ref, dinv_ref, o_ref):
    w = w_ref[...]
    xw = (jnp.dot(x_ref[...], w[:HD], preferred_element_type=jnp.float32)
          + jnp.dot(h_ref[...], w[HD:], preferred_element_type=jnp.float32))
    o_ref[...] = (dinv_ref[...] * xw)[None]


def _compute_y1(x, h, W_c1, dinv):
    return pl.pallas_call(
        _y1_body,
        grid=(NB, 2),
        in_specs=[pl.BlockSpec((BN, HD), lambda i, j: (i, 0)),
                  pl.BlockSpec((BN, HD), lambda i, j: (i, 0)),
                  pl.BlockSpec((2 * HD, 128), lambda i, j: (0, j)),
                  pl.BlockSpec((BN, 1), lambda i, j: (i, 0))],
        out_specs=pl.BlockSpec((1, BN, 128), lambda i, j: (j, i, 0)),
        out_shape=jax.ShapeDtypeStruct((2, N, 128), jnp.float32),
    )(x, h, W_c1, dinv)


def _y2_body(a_ref, y_ref, dinv_ref, b_ref, w_ref, o_ref):
    a = a_ref[...] + y_ref[...]
    full = jnp.concatenate([a[0], a[1]], axis=1)
    dinv = dinv_ref[...]
    hx = jnp.maximum(dinv * full + b_ref[...], 0.0)
    o_ref[...] = dinv * jnp.dot(hx, w_ref[...], preferred_element_type=jnp.float32)


def _compute_y2(agg1, y1, dinv, b_c1, W_mean):
    return pl.pallas_call(
        _y2_body,
        grid=(NB,),
        in_specs=[pl.BlockSpec((2, BN, 128), lambda i: (0, i, 0)),
                  pl.BlockSpec((2, BN, 128), lambda i: (0, i, 0)),
                  pl.BlockSpec((BN, 1), lambda i: (i, 0)),
                  pl.BlockSpec((1, HD), lambda i: (0, 0)),
                  pl.BlockSpec((HD, ZD), lambda i: (0, 0))],
        out_specs=pl.BlockSpec((BN, ZD), lambda i: (i, 0)),
        out_shape=jax.ShapeDtypeStruct((N, ZD), jnp.float32),
    )(agg1, y1, dinv, b_c1.reshape(1, HD), W_mean)


def _zu_body(a_ref, y2_ref, dinv_ref, bm_ref, wz_ref, bz_ref, z_ref, u_ref):
    a = a_ref[...]
    z = dinv_ref[...] * (a[0] + a[1] + y2_ref[...]) + bm_ref[...]
    z_ref[...] = z
    u_ref[...] = jnp.maximum(
        jnp.dot(z, wz_ref[...], preferred_element_type=jnp.float32)
        + bz_ref[...], 0.0)


def _compute_zu(agg2, y2, dinv, b_mean, W_phi_z, b_phi_z):
    return pl.pallas_call(
        _zu_body,
        grid=(NB,),
        in_specs=[pl.BlockSpec((2, BN, ZD), lambda i: (0, i, 0)),
                  pl.BlockSpec((BN, ZD), lambda i: (i, 0)),
                  pl.BlockSpec((BN, 1), lambda i: (i, 0)),
                  pl.BlockSpec((1, ZD), lambda i: (0, 0)),
                  pl.BlockSpec((ZD, HD), lambda i: (0, 0)),
                  pl.BlockSpec((1, HD), lambda i: (0, 0))],
        out_specs=[pl.BlockSpec((BN, ZD), lambda i: (i, 0)),
                   pl.BlockSpec((BN, HD), lambda i: (i, 0))],
        out_shape=[jax.ShapeDtypeStruct((N, ZD), jnp.float32),
                   jax.ShapeDtypeStruct((N, HD), jnp.float32)],
    )(agg2, y2, dinv, b_mean.reshape(1, ZD), W_phi_z, b_phi_z.reshape(1, HD))


def _gru_body(x_ref, u_ref, h_ref, wih_ref, bih_ref, whh_ref, bhh_ref, o_ref):
    wih = wih_ref[...]
    h = h_ref[...]
    gi = (jnp.dot(x_ref[...], wih[:HD], preferred_element_type=jnp.float32)
          + jnp.dot(u_ref[...], wih[HD:], preferred_element_type=jnp.float32)
          + bih_ref[...])
    gh = jnp.dot(h, whh_ref[...], preferred_element_type=jnp.float32) + bhh_ref[...]
    r = jax.nn.sigmoid(gi[:, :HD] + gh[:, :HD])
    zg = jax.nn.sigmoid(gi[:, HD:2 * HD] + gh[:, HD:2 * HD])
    ng = jnp.tanh(gi[:, 2 * HD:] + r * gh[:, 2 * HD:])
    o_ref[...] = (1.0 - zg) * ng + zg * h


def _compute_gru(x, u, h, W_ih, b_ih, W_hh, b_hh):
    return pl.pallas_call(
        _gru_body,
        grid=(NB,),
        in_specs=[pl.BlockSpec((BN, HD), lambda i: (i, 0)),
                  pl.BlockSpec((BN, HD), lambda i: (i, 0)),
                  pl.BlockSpec((BN, HD), lambda i: (i, 0)),
                  pl.BlockSpec((2 * HD, 3 * HD), lambda i: (0, 0)),
                  pl.BlockSpec((1, 3 * HD), lambda i: (0, 0)),
                  pl.BlockSpec((HD, 3 * HD), lambda i: (0, 0)),
                  pl.BlockSpec((1, 3 * HD), lambda i: (0, 0))],
        out_specs=pl.BlockSpec((BN, HD), lambda i: (i, 0)),
        out_shape=jax.ShapeDtypeStruct((N, HD), jnp.float32),
    )(x, u, h, W_ih, b_ih.reshape(1, 3 * HD), W_hh, b_hh.reshape(1, 3 * HD))


def kernel(xs, eis, W_phi_x, b_phi_x, W_c1, b_c1, W_mean, b_mean,
           W_phi_z, b_phi_z, W_ih, b_ih, W_hh, b_hh):
    eis = eis.astype(jnp.int32)

    x = _compute_x(xs, W_phi_x, b_phi_x)

    dinv_ts = []
    for t in range(T):
        degraw = _deg_kernel(eis[t, 1]).reshape(2, NPAD, 128)
        dinv_ts.append(_compute_dinv(degraw)[:N])

    h = jnp.zeros((N, HD), jnp.float32)
    zs = []
    for t in range(T):
        src = eis[t, 0]
        dst = eis[t, 1]
        s1_2d, d1_2d = _prep1(src, dst)
        s2_2d, d2_2d = _prep2(src, dst)
        dinv = dinv_ts[t]
        y1 = _compute_y1(x, h, W_c1, dinv)
        agg1 = _agg1(y1.reshape(2 * N, 128), s1_2d, d1_2d).reshape(2, N, 128)
        y2 = _compute_y2(agg1, y1, dinv, b_c1, W_mean)
        agg2 = _agg2(y2, s2_2d, d2_2d).reshape(2, N, 128)
        z, u = _compute_zu(agg2, y2, dinv, b_mean, W_phi_z, b_phi_z)
        h = _compute_gru(x, u, h, W_ih, b_ih, W_hh, b_hh)
        zs.append(z)
    return jnp.stack(zs)


# speedup vs baseline: 1.4014x; 1.4014x over previous
import functools

import jax
import jax.numpy as jnp
from jax import lax
from jax.experimental import pallas as pl
from jax.experimental.pallas import tpu as pltpu
from jax.experimental.pallas import tpu_sc as plsc

N = 10000
T = 3
E = 320000
XD = 128
HD = 256
ZD = 128
NPAD = 10240
DEGR = T * NPAD
BN = 400
NB = N // BN

_mesh = plsc.VectorSubcoreMesh(core_axis_name="c", subcore_axis_name="s")


@functools.partial(
    pl.kernel, mesh=_mesh,
    out_type=jax.ShapeDtypeStruct((2 * NPAD, 128), jnp.float32),
    scratch_types=[
        pltpu.VMEM((80,), jnp.int32),
        pltpu.VMEM((80, 128), jnp.float32),
        pltpu.VMEM((128, 128), jnp.float32),
        pltpu.VMEM_SHARED((NPAD, 128), jnp.float32),
    ],
)
def _deg_kernel(dst_hbm, out_hbm, didx, ones_rows, zbuf, shared):
    c = lax.axis_index("c")
    s = lax.axis_index("s")
    w = s * 2 + c

    def zrow(r, carry):
        for j in range(8):
            zbuf[r, pl.ds(16 * j, 16)] = jnp.zeros((16,), jnp.float32)
        return carry
    lax.fori_loop(0, 128, zrow, 0)

    def orow(r, carry):
        for j in range(8):
            ones_rows[r, pl.ds(16 * j, 16)] = jnp.ones((16,), jnp.float32)
        return carry
    lax.fori_loop(0, 80, orow, 0)

    def zshared(i, carry):
        k = s + i * 16
        pltpu.sync_copy(zbuf, shared.at[pl.ds(k * 128, 128)])
        return carry
    lax.fori_loop(0, NPAD // 128 // 16, zshared, 0)
    plsc.subcore_barrier()

    def chunk(k, carry):
        base = w * (E // 32) + k * 80
        pltpu.sync_copy(dst_hbm.at[pl.ds(base, 80)], didx)
        pltpu.sync_copy(ones_rows, shared.at[didx], add=True)
        return carry
    lax.fori_loop(0, (E // 32) // 80, chunk, 0)
    plsc.subcore_barrier()

    def cpout(i, carry):
        k = s + i * 16
        pltpu.sync_copy(shared.at[pl.ds(k * 128, 128)], zbuf)
        pltpu.sync_copy(zbuf, out_hbm.at[pl.ds(c * NPAD + k * 128, 128)])
        return carry
    lax.fori_loop(0, NPAD // 128 // 16, cpout, 0)


def _make_agg(split_features):
    per_tile = (E // 16) if split_features else (E // 32)
    nchunk = per_tile // 80
    CPR = 200
    NCH = N // CPR

    @functools.partial(
        pl.kernel, mesh=_mesh,
        out_type=jax.ShapeDtypeStruct((2 * N, 128), jnp.float32),
        scratch_types=[
            pltpu.VMEM((80,), jnp.int32),
            pltpu.VMEM((80,), jnp.int32),
            pltpu.VMEM((80, 128), jnp.float32),
            pltpu.VMEM((200, 128), jnp.float32),
            pltpu.VMEM_SHARED((N, 128), jnp.float32),
            pltpu.SemaphoreType.DMA,
        ],
    )
    def _agg(y_hbm, src_hbm, dst_hbm, out_hbm, sidx, didx, rows, zbuf, shared, sem):
        c = lax.axis_index("c")
        s = lax.axis_index("s")

        def zrow(r, carry):
            for j in range(8):
                zbuf[r, pl.ds(16 * j, 16)] = jnp.zeros((16,), jnp.float32)
            return carry
        lax.fori_loop(0, 200, zrow, 0)

        nmine = jnp.where(s < (NCH % 16), (NCH // 16) + 1, NCH // 16)

        def zshared(i, carry):
            k = s + i * 16
            pltpu.sync_copy(zbuf, shared.at[pl.ds(k * CPR, CPR)])
            return carry
        lax.fori_loop(0, nmine, zshared, 0)
        plsc.subcore_barrier()

        if split_features:
            ebase = s * per_tile
            sbase = c * E + ebase
        else:
            ebase = (s * 2 + c) * per_tile
            sbase = ebase

        def chunk(k, carry):
            pltpu.sync_copy(src_hbm.at[pl.ds(sbase + k * 80, 80)], sidx)
            pltpu.sync_copy(dst_hbm.at[pl.ds(ebase + k * 80, 80)], didx)
            pltpu.async_copy(y_hbm.at[sidx], rows, sem).wait()
            pltpu.sync_copy(rows, shared.at[didx], add=True)
            return carry
        lax.fori_loop(0, nchunk, chunk, 0)
        plsc.subcore_barrier()

        def cpout(i, carry):
            k = s + i * 16
            pltpu.sync_copy(shared.at[pl.ds(k * CPR, CPR)], zbuf)
            pltpu.sync_copy(zbuf, out_hbm.at[pl.ds(c * N + k * CPR, CPR)])
            return carry
        lax.fori_loop(0, nmine, cpout, 0)

    return _agg


_agg1 = _make_agg(True)
_agg2 = _make_agg(False)


def _x_body(xs_ref, w_ref, b_ref, o_ref):
    o_ref[...] = jnp.maximum(
        jnp.dot(xs_ref[...], w_ref[...], preferred_element_type=jnp.float32)
        + b_ref[...], 0.0)


def _compute_x(xs, W, b):
    return pl.pallas_call(
        _x_body,
        grid=(NB,),
        in_specs=[pl.BlockSpec((BN, XD), lambda i: (i, 0)),
                  pl.BlockSpec((XD, HD), lambda i: (0, 0)),
                  pl.BlockSpec((1, HD), lambda i: (0, 0))],
        out_specs=pl.BlockSpec((BN, HD), lambda i: (i, 0)),
        out_shape=jax.ShapeDtypeStruct((N, HD), jnp.float32),
    )(xs, W, b.reshape(1, HD))


_RB = 1280


def _dinv_body(p_ref, o_ref):
    d = p_ref[0] + p_ref[1]
    deg = 1.0 + jnp.sum(d, axis=1) * (1.0 / 128.0)
    o_ref[...] = lax.rsqrt(deg)[:, None]


def _compute_dinv(parts):
    return pl.pallas_call(
        _dinv_body,
        grid=(NPAD // _RB,),
        in_specs=[pl.BlockSpec((2, _RB, 128), lambda i: (0, i, 0))],
        out_specs=pl.BlockSpec((_RB, 1), lambda i: (i, 0)),
        out_shape=jax.ShapeDtypeStruct((NPAD, 1), jnp.float32),
    )(parts)


def _y1_body(x_ref, h_ref, w_ref, dinv_ref, o_ref):
    w = w_ref[...]
    xw = (jnp.dot(x_ref[...], w[:HD], preferred_element_type=jnp.float32)
          + jnp.dot(h_ref[...], w[HD:], preferred_element_type=jnp.float32))
    o_ref[...] = (dinv_ref[...] * xw)[None]


def _compute_y1(x, h, W_c1, dinv):
    return pl.pallas_call(
        _y1_body,
        grid=(NB, 2),
        in_specs=[pl.BlockSpec((BN, HD), lambda i, j: (i, 0)),
                  pl.BlockSpec((BN, HD), lambda i, j: (i, 0)),
                  pl.BlockSpec((2 * HD, 128), lambda i, j: (0, j)),
                  pl.BlockSpec((BN, 1), lambda i, j: (i, 0))],
        out_specs=pl.BlockSpec((1, BN, 128), lambda i, j: (j, i, 0)),
        out_shape=jax.ShapeDtypeStruct((2, N, 128), jnp.float32),
    )(x, h, W_c1, dinv)


def _y2_body(a_ref, y_ref, dinv_ref, b_ref, w_ref, o_ref):
    a = a_ref[...] + y_ref[...]
    full = jnp.concatenate([a[0], a[1]], axis=1)
    dinv = dinv_ref[...]
    hx = jnp.maximum(dinv * full + b_ref[...], 0.0)
    o_ref[...] = dinv * jnp.dot(hx, w_ref[...], preferred_element_type=jnp.float32)


def _compute_y2(agg1, y1, dinv, b_c1, W_mean):
    return pl.pallas_call(
        _y2_body,
        grid=(NB,),
        in_specs=[pl.BlockSpec((2, BN, 128), lambda i: (0, i, 0)),
                  pl.BlockSpec((2, BN, 128), lambda i: (0, i, 0)),
                  pl.BlockSpec((BN, 1), lambda i: (i, 0)),
                  pl.BlockSpec((1, HD), lambda i: (0, 0)),
                  pl.BlockSpec((HD, ZD), lambda i: (0, 0))],
        out_specs=pl.BlockSpec((BN, ZD), lambda i: (i, 0)),
        out_shape=jax.ShapeDtypeStruct((N, ZD), jnp.float32),
    )(agg1, y1, dinv, b_c1.reshape(1, HD), W_mean)


def _zu_body(a_ref, y2_ref, dinv_ref, bm_ref, wz_ref, bz_ref, z_ref, u_ref):
    a = a_ref[...]
    z = dinv_ref[...] * (a[0] + a[1] + y2_ref[...]) + bm_ref[...]
    z_ref[...] = z
    u_ref[...] = jnp.maximum(
        jnp.dot(z, wz_ref[...], preferred_element_type=jnp.float32)
        + bz_ref[...], 0.0)


def _compute_zu(agg2, y2, dinv, b_mean, W_phi_z, b_phi_z):
    return pl.pallas_call(
        _zu_body,
        grid=(NB,),
        in_specs=[pl.BlockSpec((2, BN, ZD), lambda i: (0, i, 0)),
                  pl.BlockSpec((BN, ZD), lambda i: (i, 0)),
                  pl.BlockSpec((BN, 1), lambda i: (i, 0)),
                  pl.BlockSpec((1, ZD), lambda i: (0, 0)),
                  pl.BlockSpec((ZD, HD), lambda i: (0, 0)),
                  pl.BlockSpec((1, HD), lambda i: (0, 0))],
        out_specs=[pl.BlockSpec((BN, ZD), lambda i: (i, 0)),
                   pl.BlockSpec((BN, HD), lambda i: (i, 0))],
        out_shape=[jax.ShapeDtypeStruct((N, ZD), jnp.float32),
                   jax.ShapeDtypeStruct((N, HD), jnp.float32)],
    )(agg2, y2, dinv, b_mean.reshape(1, ZD), W_phi_z, b_phi_z.reshape(1, HD))


def _gru_body(x_ref, u_ref, h_ref, wih_ref, bih_ref, whh_ref, bhh_ref, o_ref):
    wih = wih_ref[...]
    h = h_ref[...]
    gi = (jnp.dot(x_ref[...], wih[:HD], preferred_element_type=jnp.float32)
          + jnp.dot(u_ref[...], wih[HD:], preferred_element_type=jnp.float32)
          + bih_ref[...])
    gh = jnp.dot(h, whh_ref[...], preferred_element_type=jnp.float32) + bhh_ref[...]
    r = jax.nn.sigmoid(gi[:, :HD] + gh[:, :HD])
    zg = jax.nn.sigmoid(gi[:, HD:2 * HD] + gh[:, HD:2 * HD])
    ng = jnp.tanh(gi[:, 2 * HD:] + r * gh[:, 2 * HD:])
    o_ref[...] = (1.0 - zg) * ng + zg * h


def _compute_gru(x, u, h, W_ih, b_ih, W_hh, b_hh):
    return pl.pallas_call(
        _gru_body,
        grid=(NB,),
        in_specs=[pl.BlockSpec((BN, HD), lambda i: (i, 0)),
                  pl.BlockSpec((BN, HD), lambda i: (i, 0)),
                  pl.BlockSpec((BN, HD), lambda i: (i, 0)),
                  pl.BlockSpec((2 * HD, 3 * HD), lambda i: (0, 0)),
                  pl.BlockSpec((1, 3 * HD), lambda i: (0, 0)),
                  pl.BlockSpec((HD, 3 * HD), lambda i: (0, 0)),
                  pl.BlockSpec((1, 3 * HD), lambda i: (0, 0))],
        out_specs=pl.BlockSpec((BN, HD), lambda i: (i, 0)),
        out_shape=jax.ShapeDtypeStruct((N, HD), jnp.float32),
    )(x, u, h, W_ih, b_ih.reshape(1, 3 * HD), W_hh, b_hh.reshape(1, 3 * HD))


def kernel(xs, eis, W_phi_x, b_phi_x, W_c1, b_c1, W_mean, b_mean,
           W_phi_z, b_phi_z, W_ih, b_ih, W_hh, b_hh):
    eis = eis.astype(jnp.int32)

    x = _compute_x(xs, W_phi_x, b_phi_x)

    dinv_ts = []
    for t in range(T):
        degraw = _deg_kernel(eis[t, 1]).reshape(2, NPAD, 128)
        dinv_ts.append(_compute_dinv(degraw)[:N])

    h = jnp.zeros((N, HD), jnp.float32)
    zs = []
    for t in range(T):
        src = eis[t, 0]
        dst = eis[t, 1]
        src2 = jnp.concatenate([src, src + N])
        dinv = dinv_ts[t]
        y1 = _compute_y1(x, h, W_c1, dinv)
        agg1 = _agg1(y1.reshape(2 * N, 128), src2, dst).reshape(2, N, 128)
        y2 = _compute_y2(agg1, y1, dinv, b_c1, W_mean)
        agg2 = _agg2(y2, src, dst).reshape(2, N, 128)
        z, u = _compute_zu(agg2, y2, dinv, b_mean, W_phi_z, b_phi_z)
        h = _compute_gru(x, u, h, W_ih, b_ih, W_hh, b_hh)
        zs.append(z)
    return jnp.stack(zs)


# speedup vs baseline: 1.6229x; 1.1581x over previous
import functools

import jax
import jax.numpy as jnp
from jax import lax
from jax.experimental import pallas as pl
from jax.experimental.pallas import tpu as pltpu
from jax.experimental.pallas import tpu_sc as plsc

N = 10000
T = 3
E = 320000
XD = 128
HD = 256
ZD = 128
NPAD = 10240
DEGR = T * NPAD
BN = 400
NB = N // BN

_mesh = plsc.VectorSubcoreMesh(core_axis_name="c", subcore_axis_name="s")


@functools.partial(
    pl.kernel, mesh=_mesh,
    out_type=jax.ShapeDtypeStruct((2 * NPAD, 128), jnp.float32),
    scratch_types=[
        pltpu.VMEM((80,), jnp.int32),
        pltpu.VMEM((80, 128), jnp.float32),
        pltpu.VMEM((128, 128), jnp.float32),
        pltpu.VMEM_SHARED((NPAD, 128), jnp.float32),
    ],
)
def _deg_kernel(dst_hbm, out_hbm, didx, ones_rows, zbuf, shared):
    c = lax.axis_index("c")
    s = lax.axis_index("s")
    w = s * 2 + c

    def zrow(r, carry):
        for j in range(8):
            zbuf[r, pl.ds(16 * j, 16)] = jnp.zeros((16,), jnp.float32)
        return carry
    lax.fori_loop(0, 128, zrow, 0)

    def orow(r, carry):
        for j in range(8):
            ones_rows[r, pl.ds(16 * j, 16)] = jnp.ones((16,), jnp.float32)
        return carry
    lax.fori_loop(0, 80, orow, 0)

    def zshared(i, carry):
        k = s + i * 16
        pltpu.sync_copy(zbuf, shared.at[pl.ds(k * 128, 128)])
        return carry
    lax.fori_loop(0, NPAD // 128 // 16, zshared, 0)
    plsc.subcore_barrier()

    def chunk(k, carry):
        base = w * (E // 32) + k * 80
        pltpu.sync_copy(dst_hbm.at[pl.ds(base, 80)], didx)
        pltpu.sync_copy(ones_rows, shared.at[didx], add=True)
        return carry
    lax.fori_loop(0, (E // 32) // 80, chunk, 0)
    plsc.subcore_barrier()

    def cpout(i, carry):
        k = s + i * 16
        pltpu.sync_copy(shared.at[pl.ds(k * 128, 128)], zbuf)
        pltpu.sync_copy(zbuf, out_hbm.at[pl.ds(c * NPAD + k * 128, 128)])
        return carry
    lax.fori_loop(0, NPAD // 128 // 16, cpout, 0)


def _make_agg(split_features):
    per_tile = (E // 16) if split_features else (E // 32)
    nchunk = per_tile // 80
    CPR = 200
    NCH = N // CPR

    @functools.partial(
        pl.kernel, mesh=_mesh,
        out_type=jax.ShapeDtypeStruct((2 * N, 128), jnp.float32),
        scratch_types=[
            pltpu.VMEM((80,), jnp.int32),
            pltpu.VMEM((80,), jnp.int32),
            pltpu.VMEM((80, 128), jnp.float32),
            pltpu.VMEM((200, 128), jnp.float32),
            pltpu.VMEM_SHARED((N, 128), jnp.float32),
            pltpu.SemaphoreType.DMA,
        ],
    )
    def _agg(y_hbm, src_hbm, dst_hbm, out_hbm, sidx, didx, rows, zbuf, shared, sem):
        c = lax.axis_index("c")
        s = lax.axis_index("s")

        def zrow(r, carry):
            for j in range(8):
                zbuf[r, pl.ds(16 * j, 16)] = jnp.zeros((16,), jnp.float32)
            return carry
        lax.fori_loop(0, 200, zrow, 0)

        nmine = jnp.where(s < (NCH % 16), (NCH // 16) + 1, NCH // 16)

        def zshared(i, carry):
            k = s + i * 16
            pltpu.sync_copy(zbuf, shared.at[pl.ds(k * CPR, CPR)])
            return carry
        lax.fori_loop(0, nmine, zshared, 0)
        plsc.subcore_barrier()

        if split_features:
            ebase = s * per_tile
            sbase = c * E + ebase
        else:
            ebase = (s * 2 + c) * per_tile
            sbase = ebase

        def chunk(k, carry):
            pltpu.sync_copy(src_hbm.at[pl.ds(sbase + k * 80, 80)], sidx)
            gather = pltpu.async_copy(y_hbm.at[sidx], rows, sem)
            pltpu.sync_copy(dst_hbm.at[pl.ds(ebase + k * 80, 80)], didx)
            gather.wait()
            pltpu.sync_copy(rows, shared.at[didx], add=True)
            return carry
        lax.fori_loop(0, nchunk, chunk, 0)
        plsc.subcore_barrier()

        def cpout(i, carry):
            k = s + i * 16
            pltpu.sync_copy(shared.at[pl.ds(k * CPR, CPR)], zbuf)
            pltpu.sync_copy(zbuf, out_hbm.at[pl.ds(c * N + k * CPR, CPR)])
            return carry
        lax.fori_loop(0, nmine, cpout, 0)

    return _agg


_agg1 = _make_agg(True)
_agg2 = _make_agg(False)


def _x_body(xs_ref, w_ref, b_ref, o_ref):
    o_ref[...] = jnp.maximum(
        jnp.dot(xs_ref[...], w_ref[...], preferred_element_type=jnp.float32)
        + b_ref[...], 0.0)


def _compute_x(xs, W, b):
    return pl.pallas_call(
        _x_body,
        grid=(NB,),
        in_specs=[pl.BlockSpec((BN, XD), lambda i: (i, 0)),
                  pl.BlockSpec((XD, HD), lambda i: (0, 0)),
                  pl.BlockSpec((1, HD), lambda i: (0, 0))],
        out_specs=pl.BlockSpec((BN, HD), lambda i: (i, 0)),
        out_shape=jax.ShapeDtypeStruct((N, HD), jnp.float32),
    )(xs, W, b.reshape(1, HD))


_RB = 1280


def _dinv_body(p_ref, o_ref):
    d = p_ref[0] + p_ref[1]
    deg = 1.0 + jnp.sum(d, axis=1) * (1.0 / 128.0)
    o_ref[...] = lax.rsqrt(deg)[:, None]


def _compute_dinv(parts):
    return pl.pallas_call(
        _dinv_body,
        grid=(NPAD // _RB,),
        in_specs=[pl.BlockSpec((2, _RB, 128), lambda i: (0, i, 0))],
        out_specs=pl.BlockSpec((_RB, 1), lambda i: (i, 0)),
        out_shape=jax.ShapeDtypeStruct((NPAD, 1), jnp.float32),
    )(parts)


def _y1_body(x_ref, h_ref, w_ref, dinv_ref, o_ref):
    w = w_ref[...]
    xw = (jnp.dot(x_ref[...], w[:HD], preferred_element_type=jnp.float32)
          + jnp.dot(h_ref[...], w[HD:], preferred_element_type=jnp.float32))
    o_ref[...] = (dinv_ref[...] * xw)[None]


def _compute_y1(x, h, W_c1, dinv):
    return pl.pallas_call(
        _y1_body,
        grid=(NB, 2),
        in_specs=[pl.BlockSpec((BN, HD), lambda i, j: (i, 0)),
                  pl.BlockSpec((BN, HD), lambda i, j: (i, 0)),
                  pl.BlockSpec((2 * HD, 128), lambda i, j: (0, j)),
                  pl.BlockSpec((BN, 1), lambda i, j: (i, 0))],
        out_specs=pl.BlockSpec((1, BN, 128), lambda i, j: (j, i, 0)),
        out_shape=jax.ShapeDtypeStruct((2, N, 128), jnp.float32),
    )(x, h, W_c1, dinv)


def _y2_body(a_ref, y_ref, dinv_ref, b_ref, w_ref, o_ref):
    a = a_ref[...] + y_ref[...]
    full = jnp.concatenate([a[0], a[1]], axis=1)
    dinv = dinv_ref[...]
    hx = jnp.maximum(dinv * full + b_ref[...], 0.0)
    o_ref[...] = dinv * jnp.dot(hx, w_ref[...], preferred_element_type=jnp.float32)


def _compute_y2(agg1, y1, dinv, b_c1, W_mean):
    return pl.pallas_call(
        _y2_body,
        grid=(NB,),
        in_specs=[pl.BlockSpec((2, BN, 128), lambda i: (0, i, 0)),
                  pl.BlockSpec((2, BN, 128), lambda i: (0, i, 0)),
                  pl.BlockSpec((BN, 1), lambda i: (i, 0)),
                  pl.BlockSpec((1, HD), lambda i: (0, 0)),
                  pl.BlockSpec((HD, ZD), lambda i: (0, 0))],
        out_specs=pl.BlockSpec((BN, ZD), lambda i: (i, 0)),
        out_shape=jax.ShapeDtypeStruct((N, ZD), jnp.float32),
    )(agg1, y1, dinv, b_c1.reshape(1, HD), W_mean)


def _zu_body(a_ref, y2_ref, dinv_ref, bm_ref, wz_ref, bz_ref, z_ref, u_ref):
    a = a_ref[...]
    z = dinv_ref[...] * (a[0] + a[1] + y2_ref[...]) + bm_ref[...]
    z_ref[...] = z
    u_ref[...] = jnp.maximum(
        jnp.dot(z, wz_ref[...], preferred_element_type=jnp.float32)
        + bz_ref[...], 0.0)


def _compute_zu(agg2, y2, dinv, b_mean, W_phi_z, b_phi_z):
    return pl.pallas_call(
        _zu_body,
        grid=(NB,),
        in_specs=[pl.BlockSpec((2, BN, ZD), lambda i: (0, i, 0)),
                  pl.BlockSpec((BN, ZD), lambda i: (i, 0)),
                  pl.BlockSpec((BN, 1), lambda i: (i, 0)),
                  pl.BlockSpec((1, ZD), lambda i: (0, 0)),
                  pl.BlockSpec((ZD, HD), lambda i: (0, 0)),
                  pl.BlockSpec((1, HD), lambda i: (0, 0))],
        out_specs=[pl.BlockSpec((BN, ZD), lambda i: (i, 0)),
                   pl.BlockSpec((BN, HD), lambda i: (i, 0))],
        out_shape=[jax.ShapeDtypeStruct((N, ZD), jnp.float32),
                   jax.ShapeDtypeStruct((N, HD), jnp.float32)],
    )(agg2, y2, dinv, b_mean.reshape(1, ZD), W_phi_z, b_phi_z.reshape(1, HD))


def _gru_body(x_ref, u_ref, h_ref, wih_ref, bih_ref, whh_ref, bhh_ref, o_ref):
    wih = wih_ref[...]
    h = h_ref[...]
    gi = (jnp.dot(x_ref[...], wih[:HD], preferred_element_type=jnp.float32)
          + jnp.dot(u_ref[...], wih[HD:], preferred_element_type=jnp.float32)
          + bih_ref[...])
    gh = jnp.dot(h, whh_ref[...], preferred_element_type=jnp.float32) + bhh_ref[...]
    r = jax.nn.sigmoid(gi[:, :HD] + gh[:, :HD])
    zg = jax.nn.sigmoid(gi[:, HD:2 * HD] + gh[:, HD:2 * HD])
    ng = jnp.tanh(gi[:, 2 * HD:] + r * gh[:, 2 * HD:])
    o_ref[...] = (1.0 - zg) * ng + zg * h


def _compute_gru(x, u, h, W_ih, b_ih, W_hh, b_hh):
    return pl.pallas_call(
        _gru_body,
        grid=(NB,),
        in_specs=[pl.BlockSpec((BN, HD), lambda i: (i, 0)),
                  pl.BlockSpec((BN, HD), lambda i: (i, 0)),
                  pl.BlockSpec((BN, HD), lambda i: (i, 0)),
                  pl.BlockSpec((2 * HD, 3 * HD), lambda i: (0, 0)),
                  pl.BlockSpec((1, 3 * HD), lambda i: (0, 0)),
                  pl.BlockSpec((HD, 3 * HD), lambda i: (0, 0)),
                  pl.BlockSpec((1, 3 * HD), lambda i: (0, 0))],
        out_specs=pl.BlockSpec((BN, HD), lambda i: (i, 0)),
        out_shape=jax.ShapeDtypeStruct((N, HD), jnp.float32),
    )(x, u, h, W_ih, b_ih.reshape(1, 3 * HD), W_hh, b_hh.reshape(1, 3 * HD))


def kernel(xs, eis, W_phi_x, b_phi_x, W_c1, b_c1, W_mean, b_mean,
           W_phi_z, b_phi_z, W_ih, b_ih, W_hh, b_hh):
    eis = eis.astype(jnp.int32)

    x = _compute_x(xs, W_phi_x, b_phi_x)

    dinv_ts = []
    for t in range(T):
        degraw = _deg_kernel(eis[t, 1]).reshape(2, NPAD, 128)
        dinv_ts.append(_compute_dinv(degraw)[:N])

    h = jnp.zeros((N, HD), jnp.float32)
    zs = []
    for t in range(T):
        src = eis[t, 0]
        dst = eis[t, 1]
        src2 = jnp.concatenate([src, src + N])
        dinv = dinv_ts[t]
        y1 = _compute_y1(x, h, W_c1, dinv)
        agg1 = _agg1(y1.reshape(2 * N, 128), src2, dst).reshape(2, N, 128)
        y2 = _compute_y2(agg1, y1, dinv, b_c1, W_mean)
        agg2 = _agg2(y2, src, dst).reshape(2, N, 128)
        z, u = _compute_zu(agg2, y2, dinv, b_mean, W_phi_z, b_phi_z)
        h = _compute_gru(x, u, h, W_ih, b_ih, W_hh, b_hh)
        zs.append(z)
    return jnp.stack(zs)
